# baseline (speedup 1.0000x reference)
import functools

import jax
import jax.numpy as jnp
from jax.experimental import pallas as pl
from jax.experimental.pallas import tpu as pltpu
from jax.experimental.pallas import tpu_sc as plsc

DIM = 1024
HIDDEN = 4096
NUM_EXPERTS = 8
TOP_K = 2
B = 4
S = 2048

TS = 512
NS = S // TS
HC = 2048
NC = HIDDEN // HC


def _routing_kernel(x_ref, wg_ref, bg_ref, wn_ref, bn_ref, gn_ref,
                    noisy_ref, ri_ref):
  b = pl.program_id(0)
  ri_ref[pl.ds(b, 1), :] = jnp.sum(x_ref[0], axis=0, keepdims=True) / S

  @pl.when(b == B - 1)
  def _():
    ri = ri_ref[...]
    logits = jnp.dot(ri, wg_ref[...],
                     preferred_element_type=jnp.float32) + bg_ref[...]
    nlog = jnp.dot(ri, wn_ref[...],
                   preferred_element_type=jnp.float32) + bn_ref[...]
    sp = jnp.logaddexp(nlog, 0.0)
    noisy_ref[...] = logits + gn_ref[...] * sp


def _sc_route_kernel(noisy_hbm, ints_hbm, wts_hbm, nz_v, iv, wv):
  core = jax.lax.axis_index("c")
  sub = jax.lax.axis_index("s")

  @pl.when(jnp.logical_and(core == 0, sub == 0))
  def _():
    pltpu.sync_copy(noisy_hbm, nz_v)
    iota = jax.lax.broadcasted_iota(jnp.int32, (16,), 0)
    lane_e = jax.lax.rem(iota, NUM_EXPERTS)
    seg = jax.lax.div(iota, NUM_EXPERTS)
    neg = jnp.float32(-3e38)

    es, ms = [], []
    for r in range(B):
      v = nz_v[0:16] if r < 2 else nz_v[16:32]
      m = seg == (r % 2)
      masked = jnp.where(m, v, neg)
      m0 = jnp.max(masked)
      i0 = jnp.min(jnp.where(jnp.logical_and(m, masked == m0), lane_e,
                             NUM_EXPERTS))
      m2 = jnp.logical_and(m, lane_e != i0)
      masked2 = jnp.where(m2, v, neg)
      m1 = jnp.max(masked2)
      i1 = jnp.min(jnp.where(jnp.logical_and(m2, masked2 == m1), lane_e,
                             NUM_EXPERTS))
      es += [i0, i1]
      ms += [(m0, m1)]

    dvec = jnp.zeros((16,), jnp.float32)
    for r in range(B):
      m0, m1 = ms[r]
      dvec = jnp.where(iota == 2 * r, m1 - m0, dvec)
      dvec = jnp.where(iota == 2 * r + 1, m0 - m1, dvec)
    wvec = 1.0 / (1.0 + jnp.exp(dvec))

    fe, fb = [es[0]], [jnp.int32(0)]
    for rp in range(1, B * TOP_K):
      dup = jnp.zeros((), jnp.bool_)
      for rq in range(rp):
        dup = jnp.logical_or(dup, es[rq] == es[rp])
      fe.append(jnp.where(dup, fe[rp - 1], es[rp]))
      fb.append(jnp.where(dup, fb[rp - 1], jnp.int32(rp // TOP_K)))

    ivec0 = jnp.zeros((16,), jnp.int32)
    ivec1 = jnp.zeros((16,), jnp.int32)
    for j in range(B * TOP_K):
      ivec0 = jnp.where(iota == j, es[j], ivec0)
      ivec0 = jnp.where(iota == NUM_EXPERTS + j, fe[j], ivec0)
      ivec1 = jnp.where(iota == j, fb[j], ivec1)
    iv[0:16] = ivec0
    iv[16:32] = ivec1
    wv[...] = wvec
    pltpu.sync_copy(iv, ints_hbm)
    pltpu.sync_copy(wv, wts_hbm)


def _ffn_kernel(idx_ref, fe_ref, fb_ref, wts_ref, x_ref, w1_ref, w2_ref, b1_ref, b2_ref,
                out_ref):
  b = pl.program_id(0)
  k = pl.program_id(1)
  c = pl.program_id(2)
  s = pl.program_id(3)

  e = idx_ref[b, k]
  dup = jnp.zeros((), dtype=jnp.bool_)
  for rp in range(B * TOP_K):
    bp, kp = rp // TOP_K, rp % TOP_K
    earlier = rp < b * TOP_K + k
    dup = jnp.logical_or(dup,
                         jnp.logical_and(earlier, idx_ref[bp, kp] == e))
  keep = jnp.logical_not(dup)

  row = pl.ds(s * TS, TS)
  first = jnp.logical_and(k == 0, c == 0)

  @pl.when(jnp.logical_and(first, jnp.logical_not(keep)))
  def _():
    out_ref[0, row, :] = jnp.zeros((TS, DIM), dtype=jnp.float32)

  @pl.when(keep)
  def _():
    w = wts_ref[b, k]
    h = jnp.dot(x_ref[0], w1_ref[0],
                preferred_element_type=jnp.float32) + b1_ref[0]
    h = 0.5 * h * (1.0 + jax.lax.erf(h * 0.7071067811865476))
    yc = jnp.dot(h, w2_ref[0],
                 preferred_element_type=jnp.float32)
    contrib = jnp.where(c == 0, yc + b2_ref[0], yc) * w

    @pl.when(first)
    def _():
      out_ref[0, row, :] = contrib

    @pl.when(jnp.logical_not(first))
    def _():
      out_ref[0, row, :] = out_ref[0, row, :] + contrib


@jax.jit
def kernel(router_input, x, Wg, bg, Wn, bn, W1, b1, W2, b2):
  gnoise = jax.random.normal(jax.random.key(42), (B, NUM_EXPERTS),
                             dtype=jnp.float32)

  noisy = pl.pallas_call(
      _routing_kernel,
      grid=(B,),
      in_specs=[
          pl.BlockSpec((1, S, DIM), lambda b: (b, 0, 0)),
          pl.BlockSpec((DIM, NUM_EXPERTS), lambda b: (0, 0)),
          pl.BlockSpec((1, NUM_EXPERTS), lambda b: (0, 0)),
          pl.BlockSpec((DIM, NUM_EXPERTS), lambda b: (0, 0)),
          pl.BlockSpec((1, NUM_EXPERTS), lambda b: (0, 0)),
          pl.BlockSpec((B, NUM_EXPERTS), lambda b: (0, 0)),
      ],
      out_specs=pl.BlockSpec((B, NUM_EXPERTS), lambda b: (0, 0)),
      out_shape=jax.ShapeDtypeStruct((B, NUM_EXPERTS), jnp.float32),
      scratch_shapes=[pltpu.MemorySpace.VMEM((B, DIM), jnp.float32)],
      compiler_params=pltpu.CompilerParams(
          dimension_semantics=("arbitrary",)),
  )(router_input, Wg, bg.reshape(1, -1), Wn, bn.reshape(1, -1), gnoise)

  mesh = plsc.VectorSubcoreMesh(core_axis_name="c", subcore_axis_name="s",
                                num_cores=2, num_subcores=16)
  ints, wvec = pl.kernel(
      _sc_route_kernel,
      out_type=[
          jax.ShapeDtypeStruct((32,), jnp.int32),
          jax.ShapeDtypeStruct((16,), jnp.float32),
      ],
      mesh=mesh,
      scratch_types=[
          pltpu.MemorySpace.VMEM((32,), jnp.float32),
          pltpu.MemorySpace.VMEM((32,), jnp.int32),
          pltpu.MemorySpace.VMEM((16,), jnp.float32),
      ],
      compiler_params=pltpu.CompilerParams(needs_layout_passes=False),
  )(noisy.reshape(B * NUM_EXPERTS))
  idx = ints[0:8].reshape(B, TOP_K)
  fetch_e = ints[8:16].reshape(B, TOP_K)
  fetch_b = ints[16:24].reshape(B, TOP_K)
  wts = wvec[0:8].reshape(B, TOP_K)

  b1r = b1.reshape(NUM_EXPERTS, 1, HIDDEN)
  b2r = b2.reshape(NUM_EXPERTS, 1, DIM)

  grid_spec = pltpu.PrefetchScalarGridSpec(
      num_scalar_prefetch=3,
      grid=(B, TOP_K, NC, NS),
      in_specs=[
          pl.BlockSpec((B, TOP_K), memory_space=pltpu.SMEM),
          pl.BlockSpec((1, TS, DIM),
                       lambda b, k, c, s, idx, fe, fb: (fb[b, k], s, 0)),
          pl.BlockSpec((1, DIM, HC),
                       lambda b, k, c, s, idx, fe, fb: (fe[b, k], 0, c)),
          pl.BlockSpec((1, HC, DIM),
                       lambda b, k, c, s, idx, fe, fb: (fe[b, k], c, 0)),
          pl.BlockSpec((1, 1, HC),
                       lambda b, k, c, s, idx, fe, fb: (fe[b, k], 0, c)),
          pl.BlockSpec((1, 1, DIM),
                       lambda b, k, c, s, idx, fe, fb: (fe[b, k], 0, 0)),
      ],
      out_specs=pl.BlockSpec((1, S, DIM), lambda b, k, c, s, idx, fe, fb: (b, 0, 0)),
  )

  out = pl.pallas_call(
      _ffn_kernel,
      grid_spec=grid_spec,
      out_shape=jax.ShapeDtypeStruct((B, S, DIM), jnp.float32),
      compiler_params=pltpu.CompilerParams(
          dimension_semantics=("arbitrary",) * 4,
          vmem_limit_bytes=100 * 1024 * 1024),
  )(idx, fetch_e, fetch_b, wts, x, W1, W2, b1r, b2r)

  return out
